# Initial kernel scaffold; baseline (speedup 1.0000x reference)
import functools

import jax
import jax.numpy as jnp
from jax import lax
from jax.experimental import pallas as pl
from jax.experimental.pallas import tpu as pltpu
from jax.experimental.pallas import tpu_sc as plsc

N = 10000
E = 320000
D = 128
H = 128
G = 64

NC = 2
NS = 16
NW = NC * NS
KB = 128
EPT = E // NW
NB = (EPT + KB - 1) // KB
EPAD = NB * KB - EPT
DUMMY = N
ACC_ROWS = N + 16
NPS = N // NS
DEGW = 16

BR = 2000
NBLK = N // BR


def _sc_mesh():
    return plsc.VectorSubcoreMesh(core_axis_name="c", subcore_axis_name="s")


def _deg_body(dstp_hbm, out_hbm, idx_v, ones_v, zero_v, acc_sh):
    c = lax.axis_index("c")
    s = lax.axis_index("s")
    wid = c * NS + s

    def fill_ones(i, carry):
        ones_v[i, :] = jnp.full((DEGW,), 1.0, jnp.float32)
        return carry

    lax.fori_loop(0, KB, fill_ones, 0)

    def fill_zero(i, carry):
        zero_v[i, :] = jnp.zeros((DEGW,), jnp.float32)
        return carry

    lax.fori_loop(0, NPS, fill_zero, 0)

    pltpu.sync_copy(zero_v, acc_sh.at[pl.ds(s * NPS, NPS)])
    pltpu.sync_copy(dstp_hbm.at[wid], idx_v)
    plsc.subcore_barrier()

    def blk(j, carry):
        pltpu.sync_copy(ones_v, acc_sh.at[idx_v.at[j]], add=True)
        return carry

    lax.fori_loop(0, NB, blk, 0)
    plsc.subcore_barrier()
    pltpu.sync_copy(acc_sh.at[pl.ds(s * NPS, NPS)],
                    out_hbm.at[c, pl.ds(s * NPS, NPS)])


def _deg_call(dstp):
    return pl.kernel(
        _deg_body,
        out_type=jax.ShapeDtypeStruct((NC, N, DEGW), jnp.float32),
        mesh=_sc_mesh(),
        scratch_types=[
            pltpu.VMEM((NB, KB), jnp.int32),
            pltpu.VMEM((KB, DEGW), jnp.float32),
            pltpu.VMEM((NPS, DEGW), jnp.float32),
            pltpu.VMEM_SHARED((ACC_ROWS, DEGW), jnp.float32),
        ],
    )(dstp)


def _scat_body(hp_hbm, srcp_hbm, dstp_hbm, out_hbm,
               src_v, dst_v, rows_v, acc_sh, sem):
    c = lax.axis_index("c")
    s = lax.axis_index("s")
    wid = c * NS + s

    pltpu.sync_copy(hp_hbm.at[pl.ds(s * NPS, NPS)],
                    acc_sh.at[pl.ds(s * NPS, NPS)])
    pltpu.sync_copy(srcp_hbm.at[wid], src_v)
    pltpu.sync_copy(dstp_hbm.at[wid], dst_v)
    plsc.subcore_barrier()

    def blk(j, carry):
        pltpu.async_copy(hp_hbm.at[src_v.at[j]], rows_v, sem).wait()
        pltpu.sync_copy(rows_v, acc_sh.at[dst_v.at[j]], add=True)
        return carry

    lax.fori_loop(0, NB, blk, 0)
    plsc.subcore_barrier()
    pltpu.sync_copy(acc_sh.at[pl.ds(s * NPS, NPS)],
                    out_hbm.at[c, pl.ds(s * NPS, NPS)])


def _scat_call(hp, srcp, dstp):
    return pl.kernel(
        _scat_body,
        out_type=jax.ShapeDtypeStruct((NC, N, D), jnp.float32),
        mesh=_sc_mesh(),
        scratch_types=[
            pltpu.VMEM((NB, KB), jnp.int32),
            pltpu.VMEM((NB, KB), jnp.int32),
            pltpu.VMEM((KB, D), jnp.float32),
            pltpu.VMEM_SHARED((ACC_ROWS, D), jnp.float32),
            pltpu.SemaphoreType.DMA,
        ],
    )(hp, srcp, dstp)


def _mm_body(x_ref, w_ref, deg_ref, hp_ref):
    deg = deg_ref[0, :, 0] + deg_ref[1, :, 0] + 1.0
    dinv = lax.rsqrt(deg)
    h = jnp.dot(x_ref[...], w_ref[...], preferred_element_type=jnp.float32)
    hp_ref[...] = h * dinv[:, None]


def _mm_call(x, w, deg):
    return pl.pallas_call(
        _mm_body,
        grid=(NBLK,),
        in_specs=[
            pl.BlockSpec((BR, D), lambda i: (i, 0)),
            pl.BlockSpec((D, H), lambda i: (0, 0)),
            pl.BlockSpec((NC, BR, DEGW), lambda i: (0, i, 0)),
        ],
        out_specs=pl.BlockSpec((BR, H), lambda i: (i, 0)),
        out_shape=jax.ShapeDtypeStruct((N, H), jnp.float32),
    )(x, w, deg)


def _head_body(acc_ref, hp_ref, deg_ref, batch_ref, gamma_ref, beta_ref,
               w1_ref, b1_ref, w2_ref, b2_ref, w3_ref, b3_ref,
               z_ref, colsum, colsq, segsum, cnt):
    i = pl.program_id(0)
    deg = deg_ref[0, :, 0] + deg_ref[1, :, 0] + 1.0
    dinv = lax.rsqrt(deg)
    rows = (acc_ref[0] + acc_ref[1] - hp_ref[...]) * dinv[:, None]

    bidx = batch_ref[0, 0, :]
    onehot = (bidx[None, :] ==
              lax.broadcasted_iota(jnp.int32, (G, 1), 0)).astype(jnp.float32)
    seg = jnp.dot(onehot, rows, preferred_element_type=jnp.float32)
    c = jnp.sum(onehot, axis=1, keepdims=True)
    cs = jnp.sum(rows, axis=0, keepdims=True)
    cq = jnp.sum(rows * rows, axis=0, keepdims=True)

    @pl.when(i == 0)
    def _init():
        colsum[...] = cs
        colsq[...] = cq
        segsum[...] = seg
        cnt[...] = c

    @pl.when(i > 0)
    def _accum():
        colsum[...] += cs
        colsq[...] += cq
        segsum[...] += seg
        cnt[...] += c

    @pl.when(i == NBLK - 1)
    def _epilogue():
        mu = colsum[...] / float(N)
        var = colsq[...] / float(N) - mu * mu
        scale = gamma_ref[...] * lax.rsqrt(var + 1e-5)
        shift = beta_ref[...] - mu * scale
        pooled = segsum[...] / jnp.maximum(cnt[...], 1.0)
        pooled = pooled * scale + shift
        z1 = jnp.maximum(
            jnp.dot(pooled, w1_ref[...], preferred_element_type=jnp.float32)
            + b1_ref[...], 0.0)
        z2 = jnp.maximum(
            jnp.dot(z1, w2_ref[...], preferred_element_type=jnp.float32)
            + b2_ref[...], 0.0)
        z_ref[...] = (jnp.dot(z2, w3_ref[...],
                              preferred_element_type=jnp.float32)
                      + b3_ref[...])


def _head_call(accs, hp, deg, batch_r, gamma, beta, w1, b1, w2, b2, w3, b3):
    full = lambda shape: pl.BlockSpec(shape, lambda i: tuple(0 for _ in shape))
    return pl.pallas_call(
        _head_body,
        grid=(NBLK,),
        in_specs=[
            pl.BlockSpec((NC, BR, H), lambda i: (0, i, 0)),
            pl.BlockSpec((BR, H), lambda i: (i, 0)),
            pl.BlockSpec((NC, BR, DEGW), lambda i: (0, i, 0)),
            pl.BlockSpec((1, 1, BR), lambda i: (i, 0, 0)),
            full((1, H)), full((1, H)),
            full((H, 128)), full((1, 128)),
            full((128, 64)), full((1, 64)),
            full((64, 1)), full((1, 1)),
        ],
        out_specs=pl.BlockSpec((G, 1), lambda i: (0, 0)),
        out_shape=jax.ShapeDtypeStruct((G, 1), jnp.float32),
        scratch_shapes=[
            pltpu.VMEM((1, H), jnp.float32),
            pltpu.VMEM((1, H), jnp.float32),
            pltpu.VMEM((G, H), jnp.float32),
            pltpu.VMEM((G, 1), jnp.float32),
        ],
    )(accs, hp, deg, batch_r, gamma, beta, w1, b1, w2, b2, w3, b3)


def kernel(x, edge_index, batch, W_gcn, b_gcn, gamma, beta,
           W1, b1, W2, b2, W3, b3):
    srcw = edge_index[0].reshape(NW, EPT)
    dstw = edge_index[1].reshape(NW, EPT)
    srcp = jnp.pad(srcw, ((0, 0), (0, EPAD))).reshape(NW, NB, KB)
    dstp = jnp.pad(dstw, ((0, 0), (0, EPAD)),
                   constant_values=DUMMY).reshape(NW, NB, KB)

    deg = _deg_call(dstp)
    hp = _mm_call(x, W_gcn, deg)
    accs = _scat_call(hp, srcp, dstp)
    batch_r = batch.reshape(NBLK, 1, BR)
    z = _head_call(accs, hp, deg, batch_r,
                   gamma.reshape(1, H), beta.reshape(1, H),
                   W1, b1.reshape(1, 128), W2, b2.reshape(1, 64),
                   W3, b3.reshape(1, 1))
    return z


# speedup vs baseline: 18.1819x; 18.1819x over previous
import jax
import jax.numpy as jnp
from jax import lax
from jax.experimental import pallas as pl
from jax.experimental.pallas import tpu as pltpu
from jax.experimental.pallas import tpu_sc as plsc

N = 10000
E = 320000
D = 128
H = 128
G = 64
NP = 10240

NC = 2
NS = 16
NW = NC * NS
KB = 128
EPT = E // NW
NB = (EPT + KB - 1) // KB
EPTP = NB * KB
EPAD = EPTP - EPT
DUMMY = NP
ACC_ROWS = NP + 16
NPS = NP // NS

BR = 1280
NBLK = NP // BR


def _sc_mesh():
    return plsc.VectorSubcoreMesh(core_axis_name="c", subcore_axis_name="s")


def _deg_body(dstf_hbm, out_hbm, idx_v, deg_v):
    c = lax.axis_index("c")
    s = lax.axis_index("s")
    wid = c * NS + s

    def fz(i, carry):
        deg_v[pl.ds(i * 16, 16)] = jnp.zeros((16,), jnp.float32)
        return carry

    lax.fori_loop(0, ACC_ROWS // 16, fz, 0)
    pltpu.sync_copy(dstf_hbm.at[wid], idx_v)

    ones16 = jnp.full((16,), 1.0, jnp.float32)

    def blk(j, carry):
        v = idx_v[pl.ds(j * 16, 16)]
        plsc.addupdate_scatter(deg_v, [v], ones16)
        return carry

    lax.fori_loop(0, EPTP // 16, blk, 0)
    pltpu.sync_copy(deg_v.at[pl.ds(0, NP)], out_hbm.at[wid])


def _deg_call(dstf):
    return pl.kernel(
        _deg_body,
        out_type=jax.ShapeDtypeStruct((NW, NP), jnp.float32),
        mesh=_sc_mesh(),
        compiler_params=pltpu.CompilerParams(needs_layout_passes=False),
        scratch_types=[
            pltpu.VMEM((EPTP,), jnp.int32),
            pltpu.VMEM((ACC_ROWS,), jnp.float32),
        ],
    )(dstf)


def _scat_body(hp_hbm, srcp_hbm, dstp_hbm, out_hbm,
               src_v, dst_v, rows_v, acc_sh, sem):
    c = lax.axis_index("c")
    s = lax.axis_index("s")
    wid = c * NS + s

    pltpu.sync_copy(hp_hbm.at[pl.ds(s * NPS, NPS)],
                    acc_sh.at[pl.ds(s * NPS, NPS)])
    plsc.subcore_barrier()

    def blk(j, carry):
        pltpu.sync_copy(srcp_hbm.at[wid, j], src_v)
        pltpu.sync_copy(dstp_hbm.at[wid, j], dst_v)
        pltpu.async_copy(hp_hbm.at[src_v], rows_v, sem).wait()
        pltpu.sync_copy(rows_v, acc_sh.at[dst_v], add=True)
        return carry

    lax.fori_loop(0, NB, blk, 0)
    plsc.subcore_barrier()
    pltpu.sync_copy(acc_sh.at[pl.ds(s * NPS, NPS)],
                    out_hbm.at[c, pl.ds(s * NPS, NPS)])


def _scat_call(hp, srcp, dstp):
    return pl.kernel(
        _scat_body,
        out_type=jax.ShapeDtypeStruct((NC, NP, D), jnp.float32),
        mesh=_sc_mesh(),
        scratch_types=[
            pltpu.VMEM((KB,), jnp.int32),
            pltpu.VMEM((KB,), jnp.int32),
            pltpu.VMEM((KB, D), jnp.float32),
            pltpu.VMEM_SHARED((ACC_ROWS, D), jnp.float32),
            pltpu.SemaphoreType.DMA,
        ],
    )(hp, srcp, dstp)


def _mm_body(x_ref, w_ref, deg_ref, hp_ref):
    deg = jnp.sum(deg_ref[...], axis=0) + 1.0
    dinv = lax.rsqrt(deg)
    h = jnp.dot(x_ref[...], w_ref[...], preferred_element_type=jnp.float32)
    hp_ref[...] = h * dinv[:, None]


def _mm_call(x, w, deg):
    return pl.pallas_call(
        _mm_body,
        grid=(NBLK,),
        in_specs=[
            pl.BlockSpec((BR, D), lambda i: (i, 0)),
            pl.BlockSpec((D, H), lambda i: (0, 0)),
            pl.BlockSpec((NW, BR), lambda i: (0, i)),
        ],
        out_specs=pl.BlockSpec((BR, H), lambda i: (i, 0)),
        out_shape=jax.ShapeDtypeStruct((NP, H), jnp.float32),
    )(x, w, deg)


def _head_body(acc_ref, hp_ref, deg_ref, batch_ref, gamma_ref, beta_ref,
               w1_ref, b1_ref, w2_ref, b2_ref, w3_ref, b3_ref,
               z_ref, colsum, colsq, segsum, cnt):
    i = pl.program_id(0)
    deg = jnp.sum(deg_ref[...], axis=0) + 1.0
    dinv = lax.rsqrt(deg)
    rows = (acc_ref[0] + acc_ref[1] - hp_ref[...]) * dinv[:, None]

    bidx = batch_ref[0, 0, :]
    onehot = (bidx[None, :] ==
              lax.broadcasted_iota(jnp.int32, (G, 1), 0)).astype(jnp.float32)
    seg = jnp.dot(onehot, rows, preferred_element_type=jnp.float32)
    c = jnp.sum(onehot, axis=1, keepdims=True)
    cs = jnp.sum(rows, axis=0, keepdims=True)
    cq = jnp.sum(rows * rows, axis=0, keepdims=True)

    @pl.when(i == 0)
    def _init():
        colsum[...] = cs
        colsq[...] = cq
        segsum[...] = seg
        cnt[...] = c

    @pl.when(i > 0)
    def _accum():
        colsum[...] += cs
        colsq[...] += cq
        segsum[...] += seg
        cnt[...] += c

    @pl.when(i == NBLK - 1)
    def _epilogue():
        mu = colsum[...] / float(N)
        var = colsq[...] / float(N) - mu * mu
        scale = gamma_ref[...] * lax.rsqrt(var + 1e-5)
        shift = beta_ref[...] - mu * scale
        pooled = segsum[...] / jnp.maximum(cnt[...], 1.0)
        pooled = pooled * scale + shift
        z1 = jnp.maximum(
            jnp.dot(pooled, w1_ref[...], preferred_element_type=jnp.float32)
            + b1_ref[...], 0.0)
        z2 = jnp.maximum(
            jnp.dot(z1, w2_ref[...], preferred_element_type=jnp.float32)
            + b2_ref[...], 0.0)
        z_ref[...] = (jnp.dot(z2, w3_ref[...],
                              preferred_element_type=jnp.float32)
                      + b3_ref[...])


def _head_call(accs, hp, deg, batch_r, gamma, beta, w1, b1, w2, b2, w3, b3):
    full = lambda shape: pl.BlockSpec(shape, lambda i: tuple(0 for _ in shape))
    return pl.pallas_call(
        _head_body,
        grid=(NBLK,),
        in_specs=[
            pl.BlockSpec((NC, BR, H), lambda i: (0, i, 0)),
            pl.BlockSpec((BR, H), lambda i: (i, 0)),
            pl.BlockSpec((NW, BR), lambda i: (0, i)),
            pl.BlockSpec((1, 1, BR), lambda i: (i, 0, 0)),
            full((1, H)), full((1, H)),
            full((H, 128)), full((1, 128)),
            full((128, 64)), full((1, 64)),
            full((64, 1)), full((1, 1)),
        ],
        out_specs=pl.BlockSpec((G, 1), lambda i: (0, 0)),
        out_shape=jax.ShapeDtypeStruct((G, 1), jnp.float32),
        scratch_shapes=[
            pltpu.VMEM((1, H), jnp.float32),
            pltpu.VMEM((1, H), jnp.float32),
            pltpu.VMEM((G, H), jnp.float32),
            pltpu.VMEM((G, 1), jnp.float32),
        ],
    )(accs, hp, deg, batch_r, gamma, beta, w1, b1, w2, b2, w3, b3)


def kernel(x, edge_index, batch, W_gcn, b_gcn, gamma, beta,
           W1, b1, W2, b2, W3, b3):
    srcw = edge_index[0].reshape(NW, EPT)
    dstw = edge_index[1].reshape(NW, EPT)
    srcp = jnp.pad(srcw, ((0, 0), (0, EPAD))).reshape(NW, NB, KB)
    dstw_p = jnp.pad(dstw, ((0, 0), (0, EPAD)), constant_values=DUMMY)
    dstp = dstw_p.reshape(NW, NB, KB)
    xp = jnp.pad(x, ((0, NP - N), (0, 0)))
    batch_p = jnp.pad(batch, (0, NP - N), constant_values=G)

    deg = _deg_call(dstw_p)
    hp = _mm_call(xp, W_gcn, deg)
    accs = _scat_call(hp, srcp, dstp)
    batch_r = batch_p.reshape(NBLK, 1, BR)
    z = _head_call(accs, hp, deg, batch_r,
                   gamma.reshape(1, H), beta.reshape(1, H),
                   W1, b1.reshape(1, 128), W2, b2.reshape(1, 64),
                   W3, b3.reshape(1, 1))
    return z
